# baseline (speedup 1.0000x reference)
import functools

import jax
import jax.numpy as jnp
from jax import lax
from jax.experimental import pallas as pl
from jax.experimental.pallas import tpu as pltpu
from jax.experimental.pallas import tpu_sc as plsc

N = 10000
E = 320000
D = 128
DH = D // 2
G = 64
OUT_DIM = 16

NC = 2
NS = 16
CH = 128
NCHUNK = 160
NBUF = 5
NITER = NCHUNK // NBUF
E_PAD = NS * NCHUNK * CH
N_PAD = 10240
ROWS_PER_TILE = N_PAD // NS
DEGW = 16

_HI = jax.lax.Precision.HIGHEST

_MESH = plsc.VectorSubcoreMesh(core_axis_name="c", subcore_axis_name="s")
_SC_PARAMS = pltpu.CompilerParams(use_tc_tiling_on_sc=False)


@functools.partial(
    pl.kernel,
    out_type=jax.ShapeDtypeStruct((NC, N_PAD, DEGW), jnp.float32),
    mesh=_MESH,
    compiler_params=_SC_PARAMS,
    scratch_types=[
        pltpu.VMEM((NCHUNK, CH), jnp.int32),
        pltpu.VMEM((CH, DEGW), jnp.float32),
        pltpu.VMEM_SHARED((N_PAD, DEGW), jnp.float32),
    ],
)
def _deg_kernel(ei_hbm, ones_hbm, zcol_hbm, out_hbm, idx_v, ones_v, acc):
    c = lax.axis_index("c")
    s = lax.axis_index("s")
    pltpu.sync_copy(ei_hbm.at[c, s], idx_v)
    pltpu.sync_copy(ones_hbm, ones_v)
    sl = pl.ds(s * ROWS_PER_TILE, ROWS_PER_TILE)
    pltpu.sync_copy(zcol_hbm, acc.at[sl])
    plsc.subcore_barrier()

    def body(j, carry):
        pltpu.sync_copy(ones_v, acc.at[idx_v.at[j]], add=True)
        return carry

    lax.fori_loop(0, NCHUNK, body, 0)
    plsc.subcore_barrier()
    pltpu.sync_copy(acc.at[sl], out_hbm.at[c, sl])


@functools.partial(
    pl.kernel,
    out_type=jax.ShapeDtypeStruct((NC, N_PAD, DH), jnp.float32),
    mesh=_MESH,
    compiler_params=_SC_PARAMS,
    scratch_types=(
        [pltpu.VMEM((NCHUNK, CH), jnp.int32)] * 2
        + [pltpu.VMEM((CH, DH), jnp.float32)] * NBUF
        + [pltpu.VMEM_SHARED((N_PAD, DH), jnp.float32)]
        + [pltpu.SemaphoreType.DMA] * (2 * NBUF)
    ),
)
def _agg_kernel(h_st_hbm, ei_hbm, zrows_hbm, out_hbm, src_v, dst_v, *rest):
    c = lax.axis_index("c")
    s = lax.axis_index("s")
    bufs = rest[:NBUF]
    acc = rest[NBUF]
    gsem = rest[NBUF + 1:2 * NBUF + 1]
    ssem = rest[2 * NBUF + 1:]
    tab = h_st_hbm.at[c]
    pltpu.sync_copy(ei_hbm.at[0, s], src_v)
    pltpu.sync_copy(ei_hbm.at[1, s], dst_v)
    sl = pl.ds(s * ROWS_PER_TILE, ROWS_PER_TILE)
    pltpu.sync_copy(zrows_hbm, acc.at[sl])
    plsc.subcore_barrier()

    for b in range(NBUF):
        pltpu.async_copy(tab.at[src_v.at[b]], bufs[b], gsem[b])

    def body(i, carry):
        base = i * NBUF
        for b in range(NBUF):
            j = base + b
            pltpu.make_async_copy(tab.at[src_v.at[j]], bufs[b], gsem[b]).wait()
            pltpu.async_copy(bufs[b], acc.at[dst_v.at[j]], ssem[b], add=True)

        @pl.when(i < NITER - 1)
        def _refill():
            for b in range(NBUF):
                j = base + b
                pltpu.make_async_copy(
                    bufs[b], acc.at[dst_v.at[j]], ssem[b]).wait()
                pltpu.async_copy(tab.at[src_v.at[j + NBUF]], bufs[b], gsem[b])

        return carry

    lax.fori_loop(0, NITER, body, 0)
    last = (NITER - 1) * NBUF
    for b in range(NBUF):
        pltpu.make_async_copy(
            bufs[b], acc.at[dst_v.at[last + b]], ssem[b]).wait()
    plsc.subcore_barrier()
    pltpu.sync_copy(acc.at[sl], out_hbm.at[c, sl])


def _prep_body(deg_ref, x_ref, xs_ref, ns_ref, nd_ref, ins_ref):
    deg_out = deg_ref[0][:, :1]
    deg_in = deg_ref[1][:, :1]
    do_c = jnp.maximum(deg_out, 1.0)
    ns = lax.rsqrt(do_c)
    nd = lax.rsqrt(jnp.maximum(deg_in, 1.0))
    xs = x_ref[...] * ns
    xs_ref[0] = xs[:, :DH]
    xs_ref[1] = xs[:, DH:]
    ns_ref[...] = ns
    nd_ref[...] = nd
    ins_ref[...] = jnp.sqrt(do_c)


_prep_call = pl.pallas_call(
    _prep_body,
    out_shape=[
        jax.ShapeDtypeStruct((NC, N_PAD, DH), jnp.float32),
        jax.ShapeDtypeStruct((N_PAD, 1), jnp.float32),
        jax.ShapeDtypeStruct((N_PAD, 1), jnp.float32),
        jax.ShapeDtypeStruct((N_PAD, 1), jnp.float32),
    ],
)


def _layer_a_body(p, nd, w_ref, b_ref, gidr, h_out, s1_out, s2_out, cnt_out):
    agg = jnp.concatenate([p[0], p[1]], axis=1) * nd[...]
    h = jnp.dot(agg, w_ref[...], precision=_HI) + b_ref[...]
    h_out[...] = h
    ids_r = gidr[...]
    oh_gn = (ids_r == lax.broadcasted_iota(jnp.int32, (G, N_PAD), 0)
             ).astype(jnp.float32)
    s1_out[...] = jnp.dot(oh_gn, h, precision=_HI)
    s2_out[...] = jnp.dot(oh_gn, h * h, precision=_HI)
    cnt_out[...] = jnp.maximum(jnp.sum(oh_gn, axis=1, keepdims=True), 1.0)


_layer_a_call = pl.pallas_call(
    _layer_a_body,
    out_shape=[
        jax.ShapeDtypeStruct((N_PAD, D), jnp.float32),
        jax.ShapeDtypeStruct((G, D), jnp.float32),
        jax.ShapeDtypeStruct((G, D), jnp.float32),
        jax.ShapeDtypeStruct((G, 1), jnp.float32),
    ],
)


def _gn_coeffs(s1, s2, cnt, gw, gb, gms):
    mean_g = s1[...] / cnt[...]
    var_g = s2[...] / cnt[...] - mean_g * mean_g * gms[...] * (2.0 - gms[...])
    istd_g = 1.0 / jnp.sqrt(var_g + 1e-6)
    a_g = gw[...] * istd_g
    c_g = gb[...] - a_g * mean_g * gms[...]
    return a_g, c_g


def _oh_ng(gidc):
    ids_c = gidc[...]
    return (ids_c == lax.broadcasted_iota(jnp.int32, (N_PAD, G), 1)
            ).astype(jnp.float32)


def _layer_b1_body(h, s1, s2, cnt, gw, gb, gms, gidc, ns, hs_out):
    a_g, c_g = _gn_coeffs(s1, s2, cnt, gw, gb, gms)
    oh = _oh_ng(gidc)
    nsv = ns[...]
    a_n = jnp.dot(oh, a_g, precision=_HI) * nsv
    c_n = jnp.dot(oh, c_g, precision=_HI) * nsv
    hs = jnp.maximum(h[...] * a_n + c_n, 0.0)
    hs_out[0] = hs[:, :DH]
    hs_out[1] = hs[:, DH:]


_layer_b1_call = pl.pallas_call(
    _layer_b1_body,
    out_shape=jax.ShapeDtypeStruct((NC, N_PAD, DH), jnp.float32),
)


def _layer_b2_body(h, s1, s2, cnt, gw, gb, gms, gidc, gidr, h1s, ins,
                   lw_ref, lb_ref, out_ref):
    a_g, c_g = _gn_coeffs(s1, s2, cnt, gw, gb, gms)
    oh = _oh_ng(gidc)
    a_n = jnp.dot(oh, a_g, precision=_HI)
    c_n = jnp.dot(oh, c_g, precision=_HI)
    resid = jnp.concatenate([h1s[0], h1s[1]], axis=1) * ins[...]
    h2 = jnp.maximum(h[...] * a_n + c_n, 0.0) + resid
    ids_r = gidr[...]
    oh_gn = (ids_r == lax.broadcasted_iota(jnp.int32, (G, N_PAD), 0)
             ).astype(jnp.float32)
    pooled = jnp.dot(oh_gn, h2, precision=_HI)
    out_ref[...] = jnp.dot(pooled, lw_ref[...], precision=_HI) + lb_ref[...]


_layer_b2_call = pl.pallas_call(
    _layer_b2_body,
    out_shape=jax.ShapeDtypeStruct((G, OUT_DIM), jnp.float32),
)


def kernel(x, edge_index, graph_ids, W1, b1, gn1_w, gn1_b, gn1_ms,
           W2, b2, gn2_w, gn2_b, gn2_ms, lin_W, lin_b):
    f32 = jnp.float32
    ei = jnp.concatenate(
        [edge_index, jnp.full((2, E_PAD - E), N, jnp.int32)], axis=1
    ).reshape(2, NS, NCHUNK, CH)
    x_pad = jnp.concatenate([x, jnp.zeros((N_PAD - N, D), f32)], axis=0)
    gid = jnp.concatenate(
        [graph_ids, jnp.full((N_PAD - N,), -1, jnp.int32)])
    gidc = gid[:, None]
    gidr = gid[None, :]
    ones_col = jnp.ones((CH, DEGW), f32)
    zcol = jnp.zeros((ROWS_PER_TILE, DEGW), f32)
    zrows = jnp.zeros((ROWS_PER_TILE, DH), f32)

    deg = _deg_kernel(ei, ones_col, zcol)
    xs, ns, nd, ins = _prep_call(deg, x_pad)

    p = _agg_kernel(xs, ei, zrows)
    h_1, s1_1, s2_1, cnt = _layer_a_call(p, nd, W1, b1.reshape(1, D), gidr)
    h1s = _layer_b1_call(
        h_1, s1_1, s2_1, cnt, gn1_w.reshape(1, D), gn1_b.reshape(1, D),
        gn1_ms.reshape(1, D), gidc, ns)

    p2 = _agg_kernel(h1s, ei, zrows)
    h_2, s1_2, s2_2, cnt2 = _layer_a_call(p2, nd, W2, b2.reshape(1, D), gidr)
    out = _layer_b2_call(
        h_2, s1_2, s2_2, cnt2, gn2_w.reshape(1, D), gn2_b.reshape(1, D),
        gn2_ms.reshape(1, D), gidc, gidr, h1s, ins, lin_W,
        lin_b.reshape(1, OUT_DIM))
    return out


# speedup vs baseline: 1.1657x; 1.1657x over previous
import functools

import jax
import jax.numpy as jnp
from jax import lax
from jax.experimental import pallas as pl
from jax.experimental.pallas import tpu as pltpu
from jax.experimental.pallas import tpu_sc as plsc

N = 10000
E = 320000
D = 128
DH = D // 2
G = 64
OUT_DIM = 16

NC = 2
NS = 16
CH = 128
NCHUNK = 160
NBUF = 5
NITER = NCHUNK // NBUF
E_PAD = NS * NCHUNK * CH
N_PAD = 10240
ROWS_PER_TILE = N_PAD // NS
DEGW = 16

_HI = jax.lax.Precision.HIGHEST

_MESH = plsc.VectorSubcoreMesh(core_axis_name="c", subcore_axis_name="s")
_SC_PARAMS = pltpu.CompilerParams(use_tc_tiling_on_sc=False)


@functools.partial(
    pl.kernel,
    out_type=jax.ShapeDtypeStruct((NC, N_PAD, DEGW), jnp.float32),
    mesh=_MESH,
    compiler_params=_SC_PARAMS,
    scratch_types=[
        pltpu.VMEM((NCHUNK, CH), jnp.int32),
        pltpu.VMEM((CH, DEGW), jnp.float32),
        pltpu.VMEM_SHARED((N_PAD, DEGW), jnp.float32),
    ],
)
def _deg_kernel(ei_hbm, ones_hbm, zcol_hbm, out_hbm, idx_v, ones_v, acc):
    c = lax.axis_index("c")
    s = lax.axis_index("s")
    pltpu.sync_copy(ei_hbm.at[c, s], idx_v)
    pltpu.sync_copy(ones_hbm, ones_v)
    sl = pl.ds(s * ROWS_PER_TILE, ROWS_PER_TILE)
    pltpu.sync_copy(zcol_hbm, acc.at[sl])
    plsc.subcore_barrier()

    def body(j, carry):
        pltpu.sync_copy(ones_v, acc.at[idx_v.at[j]], add=True)
        return carry

    lax.fori_loop(0, NCHUNK, body, 0)
    plsc.subcore_barrier()
    pltpu.sync_copy(acc.at[sl], out_hbm.at[c, sl])


@functools.partial(
    pl.kernel,
    out_type=jax.ShapeDtypeStruct((NC, N_PAD, DH), jnp.float32),
    mesh=_MESH,
    compiler_params=_SC_PARAMS,
    scratch_types=(
        [pltpu.VMEM((NCHUNK, CH), jnp.int32)] * 2
        + [pltpu.VMEM((CH, DH), jnp.float32)] * NBUF
        + [pltpu.VMEM_SHARED((N_PAD, DH), jnp.float32)]
        + [pltpu.SemaphoreType.DMA] * (2 * NBUF)
    ),
)
def _agg_kernel(h_st_hbm, ei_hbm, zrows_hbm, out_hbm, src_v, dst_v, *rest):
    c = lax.axis_index("c")
    s = lax.axis_index("s")
    bufs = rest[:NBUF]
    acc = rest[NBUF]
    gsem = rest[NBUF + 1:2 * NBUF + 1]
    ssem = rest[2 * NBUF + 1:]
    tab = h_st_hbm.at[c]
    pltpu.sync_copy(ei_hbm.at[0, s], src_v)
    pltpu.sync_copy(ei_hbm.at[1, s], dst_v)
    sl = pl.ds(s * ROWS_PER_TILE, ROWS_PER_TILE)
    pltpu.sync_copy(zrows_hbm, acc.at[sl])
    plsc.subcore_barrier()

    for b in range(NBUF):
        pltpu.async_copy(tab.at[src_v.at[b]], bufs[b], gsem[b])

    def body(i, carry):
        base = i * NBUF
        for b in range(NBUF):
            j = base + b
            pltpu.make_async_copy(tab.at[src_v.at[j]], bufs[b], gsem[b]).wait()
            pltpu.async_copy(bufs[b], acc.at[dst_v.at[j]], ssem[b], add=True)

        @pl.when(i < NITER - 1)
        def _refill():
            for b in range(NBUF):
                j = base + b
                pltpu.make_async_copy(
                    bufs[b], acc.at[dst_v.at[j]], ssem[b]).wait()
                pltpu.async_copy(tab.at[src_v.at[j + NBUF]], bufs[b], gsem[b])

        return carry

    lax.fori_loop(0, NITER, body, 0)
    last = (NITER - 1) * NBUF
    for b in range(NBUF):
        pltpu.make_async_copy(
            bufs[b], acc.at[dst_v.at[last + b]], ssem[b]).wait()
    plsc.subcore_barrier()
    pltpu.sync_copy(acc.at[sl], out_hbm.at[c, sl])


def _norms_a_body(do_ref, di_ref, ns_ref, nd_ref, ins_ref):
    do_c = jnp.maximum(do_ref[...], 1.0)
    ns_ref[...] = lax.rsqrt(do_c)
    nd_ref[...] = lax.rsqrt(jnp.maximum(di_ref[...], 1.0))
    ins_ref[...] = jnp.sqrt(do_c)


_norms_a_call = pl.pallas_call(
    _norms_a_body,
    out_shape=[
        jax.ShapeDtypeStruct((N_PAD // CH, CH), jnp.float32),
        jax.ShapeDtypeStruct((N_PAD // CH, CH), jnp.float32),
        jax.ShapeDtypeStruct((N_PAD // CH, CH), jnp.float32),
    ],
)


def _scale_split_body(x_ref, ns_ref, xs_ref):
    xs = x_ref[...] * ns_ref[...]
    xs_ref[0] = xs[:, :DH]
    xs_ref[1] = xs[:, DH:]


_scale_split_call = pl.pallas_call(
    _scale_split_body,
    out_shape=jax.ShapeDtypeStruct((NC, N_PAD, DH), jnp.float32),
)


def _layer_a_body(p, nd, w_ref, b_ref, gidr, h_out, s1_out, s2_out, cnt_out):
    agg = jnp.concatenate([p[0], p[1]], axis=1) * nd[...]
    h = jnp.dot(agg, w_ref[...], precision=_HI) + b_ref[...]
    h_out[...] = h
    ids_r = gidr[...]
    oh_gn = (ids_r == lax.broadcasted_iota(jnp.int32, (G, N_PAD), 0)
             ).astype(jnp.float32)
    s1_out[...] = jnp.dot(oh_gn, h, precision=_HI)
    s2_out[...] = jnp.dot(oh_gn, h * h, precision=_HI)
    cnt_out[...] = jnp.maximum(jnp.sum(oh_gn, axis=1, keepdims=True), 1.0)


_layer_a_call = pl.pallas_call(
    _layer_a_body,
    out_shape=[
        jax.ShapeDtypeStruct((N_PAD, D), jnp.float32),
        jax.ShapeDtypeStruct((G, D), jnp.float32),
        jax.ShapeDtypeStruct((G, D), jnp.float32),
        jax.ShapeDtypeStruct((G, 1), jnp.float32),
    ],
)


def _gn_coeffs(s1, s2, cnt, gw, gb, gms):
    mean_g = s1[...] / cnt[...]
    var_g = s2[...] / cnt[...] - mean_g * mean_g * gms[...] * (2.0 - gms[...])
    istd_g = 1.0 / jnp.sqrt(var_g + 1e-6)
    a_g = gw[...] * istd_g
    c_g = gb[...] - a_g * mean_g * gms[...]
    return a_g, c_g


def _oh_ng(gidc):
    ids_c = gidc[...]
    return (ids_c == lax.broadcasted_iota(jnp.int32, (N_PAD, G), 1)
            ).astype(jnp.float32)


def _layer_b1_body(h, s1, s2, cnt, gw, gb, gms, gidc, ns, hs_out):
    a_g, c_g = _gn_coeffs(s1, s2, cnt, gw, gb, gms)
    oh = _oh_ng(gidc)
    nsv = ns[...]
    a_n = jnp.dot(oh, a_g, precision=_HI) * nsv
    c_n = jnp.dot(oh, c_g, precision=_HI) * nsv
    hs = jnp.maximum(h[...] * a_n + c_n, 0.0)
    hs_out[0] = hs[:, :DH]
    hs_out[1] = hs[:, DH:]


_layer_b1_call = pl.pallas_call(
    _layer_b1_body,
    out_shape=jax.ShapeDtypeStruct((NC, N_PAD, DH), jnp.float32),
)


def _layer_b2_body(h, s1, s2, cnt, gw, gb, gms, gidc, gidr, h1s, ins,
                   lw_ref, lb_ref, out_ref):
    a_g, c_g = _gn_coeffs(s1, s2, cnt, gw, gb, gms)
    oh = _oh_ng(gidc)
    a_n = jnp.dot(oh, a_g, precision=_HI)
    c_n = jnp.dot(oh, c_g, precision=_HI)
    resid = jnp.concatenate([h1s[0], h1s[1]], axis=1) * ins[...]
    h2 = jnp.maximum(h[...] * a_n + c_n, 0.0) + resid
    ids_r = gidr[...]
    oh_gn = (ids_r == lax.broadcasted_iota(jnp.int32, (G, N_PAD), 0)
             ).astype(jnp.float32)
    pooled = jnp.dot(oh_gn, h2, precision=_HI)
    out_ref[...] = jnp.dot(pooled, lw_ref[...], precision=_HI) + lb_ref[...]


_layer_b2_call = pl.pallas_call(
    _layer_b2_body,
    out_shape=jax.ShapeDtypeStruct((G, OUT_DIM), jnp.float32),
)


def kernel(x, edge_index, graph_ids, W1, b1, gn1_w, gn1_b, gn1_ms,
           W2, b2, gn2_w, gn2_b, gn2_ms, lin_W, lin_b):
    f32 = jnp.float32
    ei = jnp.concatenate(
        [edge_index, jnp.full((2, E_PAD - E), N, jnp.int32)], axis=1
    ).reshape(2, NS, NCHUNK, CH)
    x_pad = jnp.concatenate([x, jnp.zeros((N_PAD - N, D), f32)], axis=0)
    gid = jnp.concatenate(
        [graph_ids, jnp.full((N_PAD - N,), -1, jnp.int32)])
    gidc = gid[:, None]
    gidr = gid[None, :]
    ones_col = jnp.ones((CH, DEGW), f32)
    zcol = jnp.zeros((ROWS_PER_TILE, DEGW), f32)
    zrows = jnp.zeros((ROWS_PER_TILE, DH), f32)

    deg = _deg_kernel(ei, ones_col, zcol)
    deg4 = deg[:, :, 0].reshape(NC, N_PAD // CH, CH)
    ns80, nd80, ins80 = _norms_a_call(deg4[0], deg4[1])
    ns = ns80.reshape(N_PAD, 1)
    nd = nd80.reshape(N_PAD, 1)
    ins = ins80.reshape(N_PAD, 1)
    xs = _scale_split_call(x_pad, ns)

    p = _agg_kernel(xs, ei, zrows)
    h_1, s1_1, s2_1, cnt = _layer_a_call(p, nd, W1, b1.reshape(1, D), gidr)
    h1s = _layer_b1_call(
        h_1, s1_1, s2_1, cnt, gn1_w.reshape(1, D), gn1_b.reshape(1, D),
        gn1_ms.reshape(1, D), gidc, ns)

    p2 = _agg_kernel(h1s, ei, zrows)
    h_2, s1_2, s2_2, cnt2 = _layer_a_call(p2, nd, W2, b2.reshape(1, D), gidr)
    out = _layer_b2_call(
        h_2, s1_2, s2_2, cnt2, gn2_w.reshape(1, D), gn2_b.reshape(1, D),
        gn2_ms.reshape(1, D), gidc, gidr, h1s, ins, lin_W,
        lin_b.reshape(1, OUT_DIM))
    return out
